# baseline (speedup 1.0000x reference)
import functools

import jax
import jax.numpy as jnp
from jax import lax
from jax.experimental import pallas as pl
from jax.experimental.pallas import tpu as pltpu
from jax.experimental.pallas import tpu_sc as plsc

NC = 2
NS = 16
NW = NC * NS
CH = 128
GBUF = 4
IBUF = 8
OBUF = 2
LAT = 3
K1BN = 2048


@functools.lru_cache(maxsize=None)
def _make_pack(V, D):
    grid = (V + K1BN - 1) // K1BN
    half = K1BN // 2

    def body(t_ref, o_ref):
        tt = t_ref[...].T
        o_ref[...] = jnp.concatenate([tt[:half], tt[half:]], axis=1)

    return pl.pallas_call(
        body,
        grid=(grid,),
        in_specs=[pl.BlockSpec((D, K1BN), lambda i: (0, i))],
        out_specs=pl.BlockSpec((half, 2 * D), lambda i: (i, 0)),
        out_shape=jax.ShapeDtypeStruct((grid * half, 2 * D), jnp.float32),
    )


@functools.lru_cache(maxsize=None)
def _make_unpack(H, B, D):
    BH = B // CH
    DH = D // 8

    def body(r_ref, o_ref):
        v = r_ref[0]
        tile = jnp.concatenate([v[:, :D].T, v[:, D:].T], axis=1)
        o_ref[...] = tile.reshape(DH, 8, CH)[None, :, None]

    return pl.pallas_call(
        body,
        grid=(H, BH),
        in_specs=[pl.BlockSpec((1, CH // 2, 2 * D), lambda h, b: (h, b, 0))],
        out_specs=pl.BlockSpec((1, DH, 1, 8, CH), lambda h, b: (h, 0, b, 0, 0)),
        out_shape=jax.ShapeDtypeStruct((H, DH, BH, 8, CH), jnp.float32),
    )


@functools.lru_cache(maxsize=None)
def _make_gather(VP, H, B, D):
    BH = B // CH
    nchunk = H * BH // NW
    mesh = plsc.VectorSubcoreMesh(core_axis_name="c", subcore_axis_name="s")

    @functools.partial(
        pl.kernel,
        mesh=mesh,
        out_type=jax.ShapeDtypeStruct((H, B // 2, 2 * D), jnp.float32),
        scratch_types=[
            pltpu.VMEM((IBUF, CH), jnp.int32),
            pltpu.VMEM((GBUF, CH), jnp.int32),
            pltpu.VMEM((GBUF, CH, 2 * D), jnp.float32),
            pltpu.VMEM((OBUF, CH // 2, 2 * D), jnp.float32),
            pltpu.SemaphoreType.DMA((IBUF,)),
            pltpu.SemaphoreType.DMA((GBUF,)),
            pltpu.SemaphoreType.DMA((OBUF,)),
        ],
    )
    def k(tab_hbm, x_hbm, out_hbm, idx_v, half_v, rows_v, sel_v,
          isem, gsem, wsem):
        wid = lax.axis_index("s") * NC + lax.axis_index("c")
        g0 = wid * nchunk

        def hb(t):
            g = g0 + t
            return g // BH, g - (g // BH) * BH

        def i_copy(t):
            h, bh = hb(t)
            return pltpu.make_async_copy(
                x_hbm.at[h, pl.ds(bh * CH, CH)], idx_v.at[t % IBUF],
                isem.at[t % IBUF])

        def g_copy(t):
            return pltpu.make_async_copy(
                tab_hbm.at[half_v.at[t % GBUF]], rows_v.at[t % GBUF],
                gsem.at[t % GBUF])

        def w_copy(t):
            h, bh = hb(t)
            return pltpu.make_async_copy(
                sel_v.at[t % OBUF],
                out_hbm.at[h, pl.ds(bh * (CH // 2), CH // 2)],
                wsem.at[t % OBUF])

        def sel_off(iq):
            blk = lax.shift_right_logical(iq, jnp.int32(11))
            p = iq & jnp.int32(2047)
            row = lax.shift_left(blk, jnp.int32(10)) | (p & jnp.int32(1023))
            off = lax.shift_left(
                lax.shift_right_logical(p, jnp.int32(10)) & jnp.int32(1),
                jnp.int32(6))
            return row, off

        for p in range(GBUF):
            i_copy(p).start()

        def body(t, carry):
            @pl.when(t < nchunk)
            def _():
                s = t % GBUF
                i_copy(t).wait()
                for q in range(CH // 16):
                    iq = idx_v[t % IBUF, pl.ds(16 * q, 16)]
                    row, _ = sel_off(iq)
                    half_v[s, pl.ds(16 * q, 16)] = row
                g_copy(t).start()

                @pl.when(t + GBUF < nchunk)
                def _():
                    i_copy(t + GBUF).start()

            @pl.when(t >= LAT)
            def _():
                u = t - LAT
                g_copy(u).wait()

                @pl.when(t >= LAT + OBUF)
                def _():
                    w_copy(u - OBUF).wait()

                buf = rows_v.at[u % GBUF]
                sel = sel_v.at[u % OBUF]
                for q in range(CH // 16):
                    iq = idx_v[u % IBUF, pl.ds(16 * q, 16)]
                    _, offq = sel_off(iq)
                    for l in range(16):
                        b = 16 * q + l
                        off = offq[l]
                        for c in range(D // 16):
                            sel[b % 64, pl.ds((b // 64) * D + 16 * c, 16)] = (
                                buf[b, pl.ds(off + 16 * c, 16)])
                w_copy(u).start()
            return carry

        lax.fori_loop(0, nchunk + LAT, body, 0)

        for v in range(OBUF):
            w_copy(nchunk - OBUF + v).wait()

    return k


def kernel(x, table):
    B, H = x.shape
    V, D = table.shape
    xT = x.T.astype(jnp.int32)
    tT = table.T
    tabP = _make_pack(V, D)(tT)
    selP = _make_gather(tabP.shape[0], H, B, D)(tabP, xT)
    out5 = _make_unpack(H, B, D)(selP)
    return jnp.transpose(out5, (2, 4, 0, 1, 3)).reshape(B, H, D)


# speedup vs baseline: 4.5005x; 4.5005x over previous
import functools

import jax
import jax.numpy as jnp
from jax import lax
from jax.experimental import pallas as pl
from jax.experimental.pallas import tpu as pltpu
from jax.experimental.pallas import tpu_sc as plsc

NC = 2
NS = 16
NW = NC * NS
CH = 128
GBUF = 4
IBUF = 8
OBUF = 2
LAT = 3
K1BN = 2048


@functools.lru_cache(maxsize=None)
def _make_pack(V, D):
    grid = (V + K1BN - 1) // K1BN
    half = K1BN // 2

    def body(t_ref, o_ref):
        tt = t_ref[...].T
        o_ref[...] = jnp.concatenate([tt[:half], tt[half:]], axis=1)

    return pl.pallas_call(
        body,
        grid=(grid,),
        in_specs=[pl.BlockSpec((D, K1BN), lambda i: (0, i))],
        out_specs=pl.BlockSpec((half, 2 * D), lambda i: (i, 0)),
        out_shape=jax.ShapeDtypeStruct((grid * half, 2 * D), jnp.float32),
    )


@functools.lru_cache(maxsize=None)
def _make_unpack(H, B, D):
    BH = B // CH
    DH = D // 8

    NB = 8

    def body(r_ref, o_ref):
        for nb in range(NB):
            v = r_ref[0, pl.ds(nb * (CH // 2), CH // 2)]
            tile = jnp.concatenate([v[:, :D].T, v[:, D:].T], axis=1)
            o_ref[0, :, nb] = tile.reshape(DH, 8, CH)

    return pl.pallas_call(
        body,
        grid=(H, BH // NB),
        in_specs=[pl.BlockSpec((1, NB * CH // 2, 2 * D),
                               lambda h, b: (h, b, 0))],
        out_specs=pl.BlockSpec((1, DH, NB, 8, CH),
                               lambda h, b: (h, 0, b, 0, 0)),
        out_shape=jax.ShapeDtypeStruct((H, DH, BH, 8, CH), jnp.float32),
    )


@functools.lru_cache(maxsize=None)
def _make_gather(VP, H, B, D):
    BH = B // CH
    nchunk = H * BH // NW
    mesh = plsc.VectorSubcoreMesh(core_axis_name="c", subcore_axis_name="s")

    @functools.partial(
        pl.kernel,
        mesh=mesh,
        out_type=jax.ShapeDtypeStruct((H, B, D), jnp.float32),
        scratch_types=[
            pltpu.VMEM((IBUF, CH), jnp.int32),
            pltpu.VMEM((GBUF, CH), jnp.int32),
            pltpu.VMEM((GBUF, CH, 2 * D), jnp.float32),
            pltpu.VMEM((OBUF, CH, D), jnp.float32),
            pltpu.SemaphoreType.DMA((IBUF,)),
            pltpu.SemaphoreType.DMA((GBUF,)),
            pltpu.SemaphoreType.DMA((OBUF,)),
        ],
        compiler_params=pltpu.CompilerParams(use_tc_tiling_on_sc=True),
    )
    def k(tab_hbm, x_hbm, out_hbm, idx_v, half_v, rows_v, sel_v,
          isem, gsem, wsem):
        wid = lax.axis_index("s") * NC + lax.axis_index("c")
        g0 = wid * nchunk

        def hb(t):
            g = g0 + t
            return g // BH, g - (g // BH) * BH

        def i_copy(t):
            h, bh = hb(t)
            return pltpu.make_async_copy(
                x_hbm.at[h, pl.ds(bh * CH, CH)], idx_v.at[t % IBUF],
                isem.at[t % IBUF])

        def g_copy(t):
            return pltpu.make_async_copy(
                tab_hbm.at[half_v.at[t % GBUF]], rows_v.at[t % GBUF],
                gsem.at[t % GBUF])

        def w_copy(t):
            h, bh = hb(t)
            return pltpu.make_async_copy(
                sel_v.at[t % OBUF],
                out_hbm.at[h, pl.ds(bh * CH, CH)],
                wsem.at[t % OBUF])

        def sel_off(iq):
            blk = lax.shift_right_logical(iq, jnp.int32(11))
            p = iq & jnp.int32(2047)
            row = lax.shift_left(blk, jnp.int32(10)) | (p & jnp.int32(1023))
            off = lax.shift_left(
                lax.shift_right_logical(p, jnp.int32(10)) & jnp.int32(1),
                jnp.int32(6))
            return row, off

        for p in range(GBUF):
            i_copy(p).start()

        def body(t, carry):
            @pl.when(t < nchunk)
            def _():
                s = t % GBUF
                i_copy(t).wait()
                for q in range(CH // 16):
                    iq = idx_v[t % IBUF, pl.ds(16 * q, 16)]
                    row, _ = sel_off(iq)
                    half_v[s, pl.ds(16 * q, 16)] = row
                g_copy(t).start()

                @pl.when(t + GBUF < nchunk)
                def _():
                    i_copy(t + GBUF).start()

            @pl.when(t >= LAT)
            def _():
                u = t - LAT
                g_copy(u).wait()

                @pl.when(t >= LAT + OBUF)
                def _():
                    w_copy(u - OBUF).wait()

                buf = rows_v.at[u % GBUF]
                sel = sel_v.at[u % OBUF]
                for q in range(CH // 16):
                    iq = idx_v[u % IBUF, pl.ds(16 * q, 16)]
                    _, offq = sel_off(iq)
                    for l in range(16):
                        b = 16 * q + l
                        off = offq[l]
                        for c in range(D // 16):
                            sel[b, pl.ds(16 * c, 16)] = (
                                buf[b, pl.ds(off + 16 * c, 16)])
                    del iq, offq
                w_copy(u).start()
            return carry

        lax.fori_loop(0, nchunk + LAT, body, 0)

        for v in range(OBUF):
            w_copy(nchunk - OBUF + v).wait()

    return k


def kernel(x, table):
    B, H = x.shape
    V, D = table.shape
    xT = x.T.astype(jnp.int32)
    tT = table.T
    tabP = _make_pack(V, D)(tT)
    selP = _make_gather(tabP.shape[0], H, B, D)(tabP, xT)
    return jnp.transpose(selP, (1, 0, 2))


# speedup vs baseline: 5.5092x; 1.2241x over previous
import functools

import jax
import jax.numpy as jnp
from jax import lax
from jax.experimental import pallas as pl
from jax.experimental.pallas import tpu as pltpu
from jax.experimental.pallas import tpu_sc as plsc

NC = 2
NS = 16
NW = NC * NS
CH = 128
GBUF = 5
IBUF = 10
OBUF = 2
LAT = 4
K1BN = 8192


@functools.lru_cache(maxsize=None)
def _make_pack(V, D):
    grid = (V + K1BN - 1) // K1BN
    half = K1BN // 2

    def body(t_ref, o_ref):
        tt = t_ref[...].T
        o_ref[...] = jnp.concatenate([tt[:half], tt[half:]], axis=1)

    return pl.pallas_call(
        body,
        grid=(grid,),
        in_specs=[pl.BlockSpec((D, K1BN), lambda i: (0, i))],
        out_specs=pl.BlockSpec((half, 2 * D), lambda i: (i, 0)),
        out_shape=jax.ShapeDtypeStruct((grid * half, 2 * D), jnp.float32),
    )


@functools.lru_cache(maxsize=None)
def _make_unpack(H, B, D):
    BH = B // CH
    DH = D // 8

    NB = 8

    def body(r_ref, o_ref):
        for nb in range(NB):
            v = r_ref[0, pl.ds(nb * (CH // 2), CH // 2)]
            tile = jnp.concatenate([v[:, :D].T, v[:, D:].T], axis=1)
            o_ref[0, :, nb] = tile.reshape(DH, 8, CH)

    return pl.pallas_call(
        body,
        grid=(H, BH // NB),
        in_specs=[pl.BlockSpec((1, NB * CH // 2, 2 * D),
                               lambda h, b: (h, b, 0))],
        out_specs=pl.BlockSpec((1, DH, NB, 8, CH),
                               lambda h, b: (h, 0, b, 0, 0)),
        out_shape=jax.ShapeDtypeStruct((H, DH, BH, 8, CH), jnp.float32),
    )


@functools.lru_cache(maxsize=None)
def _make_gather(VP, H, B, D):
    BH = B // CH
    nchunk = H * BH // NW
    mesh = plsc.VectorSubcoreMesh(core_axis_name="c", subcore_axis_name="s")

    @functools.partial(
        pl.kernel,
        mesh=mesh,
        out_type=jax.ShapeDtypeStruct((H, B, D), jnp.float32),
        scratch_types=[
            pltpu.VMEM((IBUF, CH), jnp.int32),
            pltpu.VMEM((GBUF, CH), jnp.int32),
            pltpu.VMEM((GBUF, CH, 2 * D), jnp.float32),
            pltpu.VMEM((OBUF, CH, D), jnp.float32),
            pltpu.SemaphoreType.DMA((IBUF,)),
            pltpu.SemaphoreType.DMA((GBUF,)),
            pltpu.SemaphoreType.DMA((OBUF,)),
        ],
        compiler_params=pltpu.CompilerParams(use_tc_tiling_on_sc=True),
    )
    def k(tab_hbm, x_hbm, out_hbm, idx_v, half_v, rows_v, sel_v,
          isem, gsem, wsem):
        wid = lax.axis_index("s") * NC + lax.axis_index("c")
        g0 = wid * nchunk

        def hb(t):
            g = g0 + t
            return g // BH, g - (g // BH) * BH

        def i_copy(t):
            h, bh = hb(t)
            return pltpu.make_async_copy(
                x_hbm.at[h, pl.ds(bh * CH, CH)], idx_v.at[t % IBUF],
                isem.at[t % IBUF])

        def g_copy(t):
            return pltpu.make_async_copy(
                tab_hbm.at[half_v.at[t % GBUF]], rows_v.at[t % GBUF],
                gsem.at[t % GBUF])

        def w_copy(t):
            h, bh = hb(t)
            return pltpu.make_async_copy(
                sel_v.at[t % OBUF],
                out_hbm.at[h, pl.ds(bh * CH, CH)],
                wsem.at[t % OBUF])

        sh = K1BN.bit_length() - 1
        dsh = D.bit_length() - 1

        def sel_off(iq):
            blk = lax.shift_right_logical(iq, jnp.int32(sh))
            p = iq & jnp.int32(K1BN - 1)
            row = lax.shift_left(blk, jnp.int32(sh - 1)) | (
                p & jnp.int32(K1BN // 2 - 1))
            off = lax.shift_left(
                lax.shift_right_logical(p, jnp.int32(sh - 1)) & jnp.int32(1),
                jnp.int32(dsh))
            return row, off

        for p in range(GBUF):
            i_copy(p).start()

        def body(t, carry):
            @pl.when(t < nchunk)
            def _():
                s = t % GBUF
                i_copy(t).wait()
                for q in range(CH // 16):
                    iq = idx_v[t % IBUF, pl.ds(16 * q, 16)]
                    row, _ = sel_off(iq)
                    half_v[s, pl.ds(16 * q, 16)] = row
                g_copy(t).start()

                @pl.when(t + GBUF < nchunk)
                def _():
                    i_copy(t + GBUF).start()

            @pl.when(t >= LAT)
            def _():
                u = t - LAT
                g_copy(u).wait()

                @pl.when(t >= LAT + OBUF)
                def _():
                    w_copy(u - OBUF).wait()

                buf = rows_v.at[u % GBUF]
                sel = sel_v.at[u % OBUF]
                for q in range(CH // 16):
                    iq = idx_v[u % IBUF, pl.ds(16 * q, 16)]
                    _, offq = sel_off(iq)
                    for l in range(16):
                        b = 16 * q + l
                        off = offq[l]
                        for c in range(D // 16):
                            sel[b, pl.ds(16 * c, 16)] = (
                                buf[b, pl.ds(off + 16 * c, 16)])
                    del iq, offq
                w_copy(u).start()
            return carry

        lax.fori_loop(0, nchunk + LAT, body, 0)

        for v in range(OBUF):
            w_copy(nchunk - OBUF + v).wait()

    return k


def kernel(x, table):
    B, H = x.shape
    V, D = table.shape
    xT = x.T.astype(jnp.int32)
    tT = table.T
    tabP = _make_pack(V, D)(tT)
    selP = _make_gather(tabP.shape[0], H, B, D)(tabP, xT)
    return jnp.transpose(selP, (1, 0, 2))


# speedup vs baseline: 5.7869x; 1.0504x over previous
import functools

import jax
import jax.numpy as jnp
from jax import lax
from jax.experimental import pallas as pl
from jax.experimental.pallas import tpu as pltpu
from jax.experimental.pallas import tpu_sc as plsc

NC = 2
NS = 16
NW = NC * NS
CH = 128
GBUF = 5
IBUF = 10
OBUF = 2
LAT = 4
K1BN = 16384


@functools.lru_cache(maxsize=None)
def _make_pack(V, D):
    grid = (V + K1BN - 1) // K1BN
    half = K1BN // 2

    def body(t_ref, o_ref):
        tt = t_ref[...].T
        o_ref[...] = jnp.concatenate([tt[:half], tt[half:]], axis=1)

    return pl.pallas_call(
        body,
        grid=(grid,),
        in_specs=[pl.BlockSpec((D, K1BN), lambda i: (0, i))],
        out_specs=pl.BlockSpec((half, 2 * D), lambda i: (i, 0)),
        out_shape=jax.ShapeDtypeStruct((grid * half, 2 * D), jnp.float32),
    )


@functools.lru_cache(maxsize=None)
def _make_unpack(H, B, D):
    BH = B // CH
    DH = D // 8

    NB = 8

    def body(r_ref, o_ref):
        for nb in range(NB):
            v = r_ref[0, pl.ds(nb * (CH // 2), CH // 2)]
            tile = jnp.concatenate([v[:, :D].T, v[:, D:].T], axis=1)
            o_ref[0, :, nb] = tile.reshape(DH, 8, CH)

    return pl.pallas_call(
        body,
        grid=(H, BH // NB),
        in_specs=[pl.BlockSpec((1, NB * CH // 2, 2 * D),
                               lambda h, b: (h, b, 0))],
        out_specs=pl.BlockSpec((1, DH, NB, 8, CH),
                               lambda h, b: (h, 0, b, 0, 0)),
        out_shape=jax.ShapeDtypeStruct((H, DH, BH, 8, CH), jnp.float32),
    )


@functools.lru_cache(maxsize=None)
def _make_gather(VP, H, B, D):
    BH = B // CH
    nchunk = H * BH // NW
    mesh = plsc.VectorSubcoreMesh(core_axis_name="c", subcore_axis_name="s")

    @functools.partial(
        pl.kernel,
        mesh=mesh,
        out_type=jax.ShapeDtypeStruct((H, B, D), jnp.float32),
        scratch_types=[
            pltpu.VMEM((IBUF, CH), jnp.int32),
            pltpu.VMEM((GBUF, CH), jnp.int32),
            pltpu.VMEM((GBUF, CH, 2 * D), jnp.float32),
            pltpu.VMEM((OBUF, CH, D), jnp.float32),
            pltpu.SemaphoreType.DMA((IBUF,)),
            pltpu.SemaphoreType.DMA((GBUF,)),
            pltpu.SemaphoreType.DMA((OBUF,)),
        ],
        compiler_params=pltpu.CompilerParams(use_tc_tiling_on_sc=True),
    )
    def k(tab_hbm, x_hbm, out_hbm, idx_v, half_v, rows_v, sel_v,
          isem, gsem, wsem):
        wid = lax.axis_index("s") * NC + lax.axis_index("c")
        g0 = wid * nchunk

        def hb(t):
            g = g0 + t
            return g // BH, g - (g // BH) * BH

        def i_copy(t):
            h, bh = hb(t)
            return pltpu.make_async_copy(
                x_hbm.at[h, pl.ds(bh * CH, CH)], idx_v.at[t % IBUF],
                isem.at[t % IBUF])

        def g_copy(t):
            return pltpu.make_async_copy(
                tab_hbm.at[half_v.at[t % GBUF]], rows_v.at[t % GBUF],
                gsem.at[t % GBUF])

        def w_copy(t):
            h, bh = hb(t)
            return pltpu.make_async_copy(
                sel_v.at[t % OBUF],
                out_hbm.at[h, pl.ds(bh * CH, CH)],
                wsem.at[t % OBUF])

        sh = K1BN.bit_length() - 1
        dsh = D.bit_length() - 1

        def sel_off(iq):
            blk = lax.shift_right_logical(iq, jnp.int32(sh))
            p = iq & jnp.int32(K1BN - 1)
            row = lax.shift_left(blk, jnp.int32(sh - 1)) | (
                p & jnp.int32(K1BN // 2 - 1))
            off = lax.shift_left(
                lax.shift_right_logical(p, jnp.int32(sh - 1)) & jnp.int32(1),
                jnp.int32(dsh))
            return row, off

        for p in range(GBUF):
            i_copy(p).start()

        def body(t, carry):
            @pl.when(t < nchunk)
            def _():
                s = t % GBUF
                i_copy(t).wait()
                for q in range(CH // 16):
                    iq = idx_v[t % IBUF, pl.ds(16 * q, 16)]
                    row, _ = sel_off(iq)
                    half_v[s, pl.ds(16 * q, 16)] = row
                g_copy(t).start()

                @pl.when(t + GBUF < nchunk)
                def _():
                    i_copy(t + GBUF).start()

            @pl.when(t >= LAT)
            def _():
                u = t - LAT
                g_copy(u).wait()

                @pl.when(t >= LAT + OBUF)
                def _():
                    w_copy(u - OBUF).wait()

                buf = rows_v.at[u % GBUF]
                sel = sel_v.at[u % OBUF]
                for q in range(CH // 16):
                    iq = idx_v[u % IBUF, pl.ds(16 * q, 16)]
                    _, offq = sel_off(iq)
                    for l in range(16):
                        b = 16 * q + l
                        off = offq[l]
                        for c in range(D // 16):
                            sel[b, pl.ds(16 * c, 16)] = (
                                buf[b, pl.ds(off + 16 * c, 16)])
                    del iq, offq
                w_copy(u).start()
            return carry

        lax.fori_loop(0, nchunk + LAT, body, 0)

        for v in range(OBUF):
            w_copy(nchunk - OBUF + v).wait()

    return k


def kernel(x, table):
    B, H = x.shape
    V, D = table.shape
    xT = x.T.astype(jnp.int32)
    tT = table.T
    tabP = _make_pack(V, D)(tT)
    selP = _make_gather(tabP.shape[0], H, B, D)(tabP, xT)
    return jnp.transpose(selP, (1, 0, 2))


# speedup vs baseline: 5.8557x; 1.0119x over previous
import functools

import jax
import jax.numpy as jnp
from jax import lax
from jax.experimental import pallas as pl
from jax.experimental.pallas import tpu as pltpu
from jax.experimental.pallas import tpu_sc as plsc

NC = 2
NS = 16
NW = NC * NS
CH = 128
GBUF = 5
IBUF = 10
OBUF = 2
LAT = 4
K1BN = 32768


@functools.lru_cache(maxsize=None)
def _make_pack(V, D):
    grid = (V + K1BN - 1) // K1BN
    half = K1BN // 2

    def body(t_ref, o_ref):
        tt = t_ref[...].T
        o_ref[...] = jnp.concatenate([tt[:half], tt[half:]], axis=1)

    return pl.pallas_call(
        body,
        grid=(grid,),
        in_specs=[pl.BlockSpec((D, K1BN), lambda i: (0, i))],
        out_specs=pl.BlockSpec((half, 2 * D), lambda i: (i, 0)),
        out_shape=jax.ShapeDtypeStruct((grid * half, 2 * D), jnp.float32),
    )


@functools.lru_cache(maxsize=None)
def _make_unpack(H, B, D):
    BH = B // CH
    DH = D // 8

    NB = 8

    def body(r_ref, o_ref):
        for nb in range(NB):
            v = r_ref[0, pl.ds(nb * (CH // 2), CH // 2)]
            tile = jnp.concatenate([v[:, :D].T, v[:, D:].T], axis=1)
            o_ref[0, :, nb] = tile.reshape(DH, 8, CH)

    return pl.pallas_call(
        body,
        grid=(H, BH // NB),
        in_specs=[pl.BlockSpec((1, NB * CH // 2, 2 * D),
                               lambda h, b: (h, b, 0))],
        out_specs=pl.BlockSpec((1, DH, NB, 8, CH),
                               lambda h, b: (h, 0, b, 0, 0)),
        out_shape=jax.ShapeDtypeStruct((H, DH, BH, 8, CH), jnp.float32),
    )


@functools.lru_cache(maxsize=None)
def _make_gather(VP, H, B, D):
    BH = B // CH
    nchunk = H * BH // NW
    mesh = plsc.VectorSubcoreMesh(core_axis_name="c", subcore_axis_name="s")

    @functools.partial(
        pl.kernel,
        mesh=mesh,
        out_type=jax.ShapeDtypeStruct((H, B, D), jnp.float32),
        scratch_types=[
            pltpu.VMEM((IBUF, CH), jnp.int32),
            pltpu.VMEM((GBUF, CH), jnp.int32),
            pltpu.VMEM((GBUF, CH, 2 * D), jnp.float32),
            pltpu.VMEM((OBUF, CH, D), jnp.float32),
            pltpu.SemaphoreType.DMA((IBUF,)),
            pltpu.SemaphoreType.DMA((GBUF,)),
            pltpu.SemaphoreType.DMA((OBUF,)),
        ],
        compiler_params=pltpu.CompilerParams(use_tc_tiling_on_sc=True),
    )
    def k(tab_hbm, x_hbm, out_hbm, idx_v, half_v, rows_v, sel_v,
          isem, gsem, wsem):
        wid = lax.axis_index("s") * NC + lax.axis_index("c")
        g0 = wid * nchunk

        def hb(t):
            g = g0 + t
            return g // BH, g - (g // BH) * BH

        def i_copy(t):
            h, bh = hb(t)
            return pltpu.make_async_copy(
                x_hbm.at[h, pl.ds(bh * CH, CH)], idx_v.at[t % IBUF],
                isem.at[t % IBUF])

        def g_copy(t):
            return pltpu.make_async_copy(
                tab_hbm.at[half_v.at[t % GBUF]], rows_v.at[t % GBUF],
                gsem.at[t % GBUF])

        def w_copy(t):
            h, bh = hb(t)
            return pltpu.make_async_copy(
                sel_v.at[t % OBUF],
                out_hbm.at[h, pl.ds(bh * CH, CH)],
                wsem.at[t % OBUF])

        sh = K1BN.bit_length() - 1
        dsh = D.bit_length() - 1

        def sel_off(iq):
            blk = lax.shift_right_logical(iq, jnp.int32(sh))
            p = iq & jnp.int32(K1BN - 1)
            row = lax.shift_left(blk, jnp.int32(sh - 1)) | (
                p & jnp.int32(K1BN // 2 - 1))
            off = lax.shift_left(
                lax.shift_right_logical(p, jnp.int32(sh - 1)) & jnp.int32(1),
                jnp.int32(dsh))
            return row, off

        for p in range(GBUF):
            i_copy(p).start()

        def body(t, carry):
            @pl.when(t < nchunk)
            def _():
                s = t % GBUF
                i_copy(t).wait()
                for q in range(CH // 16):
                    iq = idx_v[t % IBUF, pl.ds(16 * q, 16)]
                    row, _ = sel_off(iq)
                    half_v[s, pl.ds(16 * q, 16)] = row
                g_copy(t).start()

                @pl.when(t + GBUF < nchunk)
                def _():
                    i_copy(t + GBUF).start()

            @pl.when(t >= LAT)
            def _():
                u = t - LAT
                g_copy(u).wait()

                @pl.when(t >= LAT + OBUF)
                def _():
                    w_copy(u - OBUF).wait()

                buf = rows_v.at[u % GBUF]
                sel = sel_v.at[u % OBUF]
                for q in range(CH // 16):
                    iq = idx_v[u % IBUF, pl.ds(16 * q, 16)]
                    _, offq = sel_off(iq)
                    for l in range(16):
                        b = 16 * q + l
                        off = offq[l]
                        for c in range(D // 16):
                            sel[b, pl.ds(16 * c, 16)] = (
                                buf[b, pl.ds(off + 16 * c, 16)])
                    del iq, offq
                w_copy(u).start()
            return carry

        lax.fori_loop(0, nchunk + LAT, body, 0)

        for v in range(OBUF):
            w_copy(nchunk - OBUF + v).wait()

    return k


def kernel(x, table):
    B, H = x.shape
    V, D = table.shape
    xT = x.T.astype(jnp.int32)
    tT = table.T
    tabP = _make_pack(V, D)(tT)
    selP = _make_gather(tabP.shape[0], H, B, D)(tabP, xT)
    return jnp.transpose(selP, (1, 0, 2))


# speedup vs baseline: 5.8598x; 1.0007x over previous
import functools

import jax
import jax.numpy as jnp
from jax import lax
from jax.experimental import pallas as pl
from jax.experimental.pallas import tpu as pltpu
from jax.experimental.pallas import tpu_sc as plsc

NC = 2
NS = 16
NW = NC * NS
CH = 128
GBUF = 5
IBUF = 10
OBUF = 2
LAT = 4
K1BN = 32768


@functools.lru_cache(maxsize=None)
def _make_pack(V, D):
    grid = (V + K1BN - 1) // K1BN
    half = K1BN // 2

    def body(t_ref, o_ref):
        tt = t_ref[...].T
        o_ref[...] = jnp.concatenate([tt[:half], tt[half:]], axis=1)

    return pl.pallas_call(
        body,
        grid=(grid,),
        in_specs=[pl.BlockSpec((D, K1BN), lambda i: (0, i))],
        out_specs=pl.BlockSpec((half, 2 * D), lambda i: (i, 0)),
        out_shape=jax.ShapeDtypeStruct((grid * half, 2 * D), jnp.float32),
    )


@functools.lru_cache(maxsize=None)
def _make_gather(VP, H, B, D):
    BH = B // CH
    nchunk = H * BH // NW
    mesh = plsc.VectorSubcoreMesh(core_axis_name="c", subcore_axis_name="s")

    @functools.partial(
        pl.kernel,
        mesh=mesh,
        out_type=jax.ShapeDtypeStruct((H, B, D), jnp.float32),
        scratch_types=[
            pltpu.VMEM((IBUF, CH), jnp.int32),
            pltpu.VMEM((GBUF, CH), jnp.int32),
            pltpu.VMEM((GBUF, CH, 2 * D), jnp.float32),
            pltpu.VMEM((OBUF, CH, D), jnp.float32),
            pltpu.SemaphoreType.DMA((IBUF,)),
            pltpu.SemaphoreType.DMA((GBUF,)),
            pltpu.SemaphoreType.DMA((OBUF,)),
        ],
        compiler_params=pltpu.CompilerParams(use_tc_tiling_on_sc=True),
    )
    def k(tab_hbm, x_hbm, out_hbm, idx_v, half_v, rows_v, sel_v,
          isem, gsem, wsem):
        wid = lax.axis_index("s") * NC + lax.axis_index("c")
        g0 = wid * nchunk

        def hb(t):
            g = g0 + t
            return g // BH, g - (g // BH) * BH

        def i_copy(t):
            h, bh = hb(t)
            return pltpu.make_async_copy(
                x_hbm.at[h, pl.ds(bh * CH, CH)], idx_v.at[t % IBUF],
                isem.at[t % IBUF])

        def g_copy(t):
            return pltpu.make_async_copy(
                tab_hbm.at[half_v.at[t % GBUF]], rows_v.at[t % GBUF],
                gsem.at[t % GBUF])

        def w_copy(t):
            h, bh = hb(t)
            return pltpu.make_async_copy(
                sel_v.at[t % OBUF],
                out_hbm.at[h, pl.ds(bh * CH, CH)],
                wsem.at[t % OBUF])

        sh = K1BN.bit_length() - 1
        dsh = D.bit_length() - 1

        def sel_off(iq):
            blk = lax.shift_right_logical(iq, jnp.int32(sh))
            p = iq & jnp.int32(K1BN - 1)
            row = lax.shift_left(blk, jnp.int32(sh - 1)) | (
                p & jnp.int32(K1BN // 2 - 1))
            off = lax.shift_left(
                lax.shift_right_logical(p, jnp.int32(sh - 1)) & jnp.int32(1),
                jnp.int32(dsh))
            return row, off

        for p in range(GBUF):
            i_copy(p).start()

        def body(t, carry):
            @pl.when(t < nchunk)
            def _():
                s = t % GBUF
                i_copy(t).wait()
                for q in range(CH // 16):
                    iq = idx_v[t % IBUF, pl.ds(16 * q, 16)]
                    row, _ = sel_off(iq)
                    half_v[s, pl.ds(16 * q, 16)] = row
                g_copy(t).start()

                @pl.when(t + GBUF < nchunk)
                def _():
                    i_copy(t + GBUF).start()

            @pl.when(t >= LAT)
            def _():
                u = t - LAT
                g_copy(u).wait()

                @pl.when(t >= LAT + OBUF)
                def _():
                    w_copy(u - OBUF).wait()

                buf = rows_v.at[u % GBUF]
                sel = sel_v.at[u % OBUF]
                for q in range(CH // 16):
                    iq = idx_v[u % IBUF, pl.ds(16 * q, 16)]
                    _, offq = sel_off(iq)
                    for l in range(16):
                        b = 16 * q + l
                        off = offq[l]
                        for c in range(D // 16):
                            sel[b, pl.ds(16 * c, 16)] = (
                                buf[b, pl.ds(off + 16 * c, 16)])
                w_copy(u).start()
            return carry

        lax.fori_loop(0, nchunk + LAT, body, 0)

        for v in range(OBUF):
            w_copy(nchunk - OBUF + v).wait()

    return k


def kernel(x, table):
    B, H = x.shape
    V, D = table.shape
    xT = x.T.astype(jnp.int32)
    tT = table.T
    tabP = _make_pack(V, D)(tT)
    selP = _make_gather(tabP.shape[0], H, B, D)(tabP, xT)
    return jnp.transpose(selP, (1, 0, 2))
